# baseline (speedup 1.0000x reference)
import functools

import jax
import jax.numpy as jnp
from jax import lax
from jax.experimental import pallas as pl
from jax.experimental.pallas import tpu as pltpu
from jax.experimental.pallas import tpu_sc as plsc

EMBED_DIM = 384
SUB = EMBED_DIM // 3
B, N = 16, 2048
HALF = N // 2
ROWS = 256
T = N // ROWS
L = 16
CHUNKS = N // L
K3 = 3
CAND = K3 * L

_mesh = plsc.VectorSubcoreMesh(core_axis_name="c", subcore_axis_name="s")


@functools.partial(
    pl.kernel,
    out_type=jax.ShapeDtypeStruct((B, 2, HALF * CAND), jnp.float32),
    mesh=_mesh,
    scratch_types=[
        pltpu.VMEM((N,), jnp.float32),
        pltpu.VMEM((N,), jnp.float32),
        pltpu.VMEM((N,), jnp.float32),
        pltpu.VMEM((N,), jnp.float32),
        pltpu.VMEM((HALF * CAND,), jnp.float32),
    ],
)
def _knn_sc(xr_hbm, yr_hbm, zr_hbm, x2_hbm, out_hbm,
            xr_v, yr_v, zr_v, x2_v, om_v):
    half = lax.axis_index("c")
    batch = lax.axis_index("s")

    pltpu.sync_copy(xr_hbm.at[batch], xr_v)
    pltpu.sync_copy(yr_hbm.at[batch], yr_v)
    pltpu.sync_copy(zr_hbm.at[batch], zr_v)
    pltpu.sync_copy(x2_hbm.at[batch], x2_v)

    iota = lax.broadcasted_iota(jnp.int32, (L,), 0)
    inf_v = jnp.full((L,), jnp.inf, jnp.float32)
    rowbase = half * HALF

    R = 8

    def multi_proc(r_loc, cs, rr, qs):
        bs = [[jnp.full((L,), -2.0 * q, jnp.float32) for q in t] for t in qs]

        def insert(t, m1, m2, m3):
            lo = jnp.minimum(m1, t)
            hi = jnp.maximum(m1, t)
            lo2 = jnp.minimum(m2, hi)
            hi2 = jnp.maximum(m2, hi)
            return lo, lo2, jnp.minimum(m3, hi2)

        def chunk_all(ci, carry, masked):
            sl = pl.ds(ci * L, L)
            vx, vy, vz, v2 = xr_v[sl], yr_v[sl], zr_v[sl], x2_v[sl]
            out = []
            for i in range(R):
                b = bs[i]
                t = v2 + ((vx * b[0] + vy * b[1]) + vz * b[2])
                if masked:
                    t = jnp.where(iota == rr + i, inf_v, t)
                out.extend(insert(t, *carry[3 * i:3 * i + 3]))
            return tuple(out)

        def cbody(ci, carry):
            return chunk_all(ci, carry, False)

        m = lax.fori_loop(0, cs, cbody, (inf_v,) * (3 * R))
        m = chunk_all(cs, m, True)
        m = lax.fori_loop(cs + 1, CHUNKS, cbody, m)
        for i in range(R):
            off = (r_loc + i) * CAND
            for k in range(K3):
                om_v[pl.ds(off + k * L, L)] = m[3 * i + k]

    def gbody(g, _):
        lb = g * L
        sl = pl.ds(rowbase + lb, L)
        vxr, vyr, vzr = xr_v[sl], yr_v[sl], zr_v[sl]
        cs = half * (HALF // L) + g
        for rr in range(0, L, R):
            multi_proc(lb + rr, cs, rr,
                       [(vxr[rr + i], vyr[rr + i], vzr[rr + i])
                        for i in range(R)])
        return 0

    lax.fori_loop(0, HALF // L, gbody, 0)
    pltpu.sync_copy(om_v, out_hbm.at[batch, half])


def _tc_body(pts_ref, cand_ref, mrel_ref, vdist_ref, vdens_ref, cvec_ref,
             out_ref):
    t = pl.program_id(1)
    pts = pts_ref[0]
    rows = pts_ref[0, pl.ds(t * ROWS, ROWS), :]

    cen = jnp.mean(pts, axis=0, keepdims=True)
    rel = rows - cen
    cd = jnp.sqrt(jnp.sum(rel * rel, axis=1, keepdims=True))

    x2r = jnp.sum(rows * rows, axis=1, keepdims=True)
    vals = jnp.maximum(cand_ref[0] + x2r, 0.0)
    ci = lax.broadcasted_iota(jnp.int32, (ROWS, CAND), 1)
    big = jnp.int32(2**30)
    ssum = jnp.zeros((ROWS, 1), jnp.float32)
    for k in range(3):
        m = jnp.min(vals, axis=1, keepdims=True)
        ssum = ssum + jnp.sqrt(m)
        if k < 2:
            sel = jnp.where(vals == m, ci, big)
            cmin = jnp.min(sel, axis=1, keepdims=True)
            vals = jnp.where(ci == cmin, jnp.inf, vals)
    dens = ssum * (1.0 / 3.0)

    acc = cvec_ref[...] + cd * vdist_ref[...] + dens * vdens_ref[...]
    acc = acc + rel[:, 0:1] * mrel_ref[0:1, :]
    acc = acc + rel[:, 1:2] * mrel_ref[1:2, :]
    acc = acc + rel[:, 2:3] * mrel_ref[2:3, :]
    out_ref[0] = acc


def kernel(points, W_rel, b_rel, W_dist, b_dist, W_dens, b_dens, W_out, b_out):
    mrel = W_rel @ W_out[:SUB]
    vdist = W_dist @ W_out[SUB:2 * SUB]
    vdens = W_dens @ W_out[2 * SUB:]
    cvec = (b_rel @ W_out[:SUB] + b_dist @ W_out[SUB:2 * SUB]
            + b_dens @ W_out[2 * SUB:] + b_out)[None, :]

    pts_t = jnp.transpose(points, (0, 2, 1))
    pts_r = lax.reduce_precision(pts_t, exponent_bits=8, mantissa_bits=7)
    xr, yr, zr = pts_r[:, 0], pts_r[:, 1], pts_r[:, 2]
    x2 = jnp.sum(pts_t * pts_t, axis=1)

    cand_raw = _knn_sc(xr, yr, zr, x2)
    cand = cand_raw.reshape(B, N, CAND)

    return pl.pallas_call(
        _tc_body,
        grid=(B, T),
        in_specs=[
            pl.BlockSpec((1, N, 3), lambda b, t: (b, 0, 0)),
            pl.BlockSpec((1, ROWS, CAND), lambda b, t: (b, t, 0)),
            pl.BlockSpec((3, EMBED_DIM), lambda b, t: (0, 0)),
            pl.BlockSpec((1, EMBED_DIM), lambda b, t: (0, 0)),
            pl.BlockSpec((1, EMBED_DIM), lambda b, t: (0, 0)),
            pl.BlockSpec((1, EMBED_DIM), lambda b, t: (0, 0)),
        ],
        out_specs=pl.BlockSpec((1, ROWS, EMBED_DIM), lambda b, t: (b, t, 0)),
        out_shape=jax.ShapeDtypeStruct((B, N, EMBED_DIM), jnp.float32),
    )(points, cand, mrel, vdist, vdens, cvec)


# speedup vs baseline: 1.5181x; 1.5181x over previous
import functools

import jax
import jax.numpy as jnp
from jax import lax
from jax.experimental import pallas as pl
from jax.experimental.pallas import tpu as pltpu
from jax.experimental.pallas import tpu_sc as plsc

EMBED_DIM = 384
SUB = EMBED_DIM // 3
B, N = 16, 2048
HALF = N // 2
ROWS = 256
T = N // ROWS
L = 16
CHUNKS = N // L
K3 = 3
CAND = K3 * L

_mesh = plsc.VectorSubcoreMesh(core_axis_name="c", subcore_axis_name="s")


@functools.partial(
    pl.kernel,
    out_type=jax.ShapeDtypeStruct((B, 2, HALF * CAND), jnp.float32),
    mesh=_mesh,
    scratch_types=[
        pltpu.VMEM((N,), jnp.float32),
        pltpu.VMEM((N,), jnp.float32),
        pltpu.VMEM((N,), jnp.float32),
        pltpu.VMEM((N,), jnp.float32),
        pltpu.VMEM((HALF * CAND,), jnp.float32),
    ],
)
def _knn_sc(xr_hbm, yr_hbm, zr_hbm, x2_hbm, out_hbm,
            xr_v, yr_v, zr_v, x2_v, om_v):
    half = lax.axis_index("c")
    batch = lax.axis_index("s")

    pltpu.sync_copy(xr_hbm.at[batch], xr_v)
    pltpu.sync_copy(yr_hbm.at[batch], yr_v)
    pltpu.sync_copy(zr_hbm.at[batch], zr_v)
    pltpu.sync_copy(x2_hbm.at[batch], x2_v)

    iota = lax.broadcasted_iota(jnp.int32, (L,), 0)
    inf_v = jnp.full((L,), jnp.inf, jnp.float32)
    rowbase = half * HALF

    R = 4

    def multi_proc(r_loc, cs, rr, qs):
        bs = [[jnp.full((L,), -2.0 * q, jnp.float32) for q in t] for t in qs]

        def insert(t, m1, m2, m3):
            lo = jnp.minimum(m1, t)
            hi = jnp.maximum(m1, t)
            lo2 = jnp.minimum(m2, hi)
            hi2 = jnp.maximum(m2, hi)
            return lo, lo2, jnp.minimum(m3, hi2)

        def chunk_all(ci, carry, masked):
            sl = pl.ds(ci * L, L)
            vx, vy, vz, v2 = xr_v[sl], yr_v[sl], zr_v[sl], x2_v[sl]
            out = []
            for i in range(R):
                b = bs[i]
                t = v2 + ((vx * b[0] + vy * b[1]) + vz * b[2])
                if masked:
                    t = jnp.where(iota == rr + i, inf_v, t)
                out.extend(insert(t, *carry[3 * i:3 * i + 3]))
            return tuple(out)

        def cbody(ci, carry):
            return chunk_all(ci, carry, False)

        m = lax.fori_loop(0, cs, cbody, (inf_v,) * (3 * R))
        m = chunk_all(cs, m, True)
        m = lax.fori_loop(cs + 1, CHUNKS, cbody, m)
        for i in range(R):
            off = (r_loc + i) * CAND
            for k in range(K3):
                om_v[pl.ds(off + k * L, L)] = m[3 * i + k]

    def gbody(g, _):
        lb = g * L
        sl = pl.ds(rowbase + lb, L)
        vxr, vyr, vzr = xr_v[sl], yr_v[sl], zr_v[sl]
        cs = half * (HALF // L) + g
        for rr in range(0, L, R):
            multi_proc(lb + rr, cs, rr,
                       [(vxr[rr + i], vyr[rr + i], vzr[rr + i])
                        for i in range(R)])
        return 0

    lax.fori_loop(0, HALF // L, gbody, 0)
    pltpu.sync_copy(om_v, out_hbm.at[batch, half])


def _tc_body(pts_ref, cand_ref, mrel_ref, vdist_ref, vdens_ref, cvec_ref,
             out_ref):
    rows = pts_ref[0]

    cen = jnp.mean(rows, axis=0, keepdims=True)
    rel = rows - cen
    cd = jnp.sqrt(jnp.sum(rel * rel, axis=1, keepdims=True))

    x2r = jnp.sum(rows * rows, axis=1, keepdims=True)
    vals = jnp.maximum(cand_ref[0] + x2r, 0.0)
    ci = lax.broadcasted_iota(jnp.int32, (N, CAND), 1).astype(jnp.float32)
    big = jnp.float32(1e9)
    ssum = jnp.zeros((N, 1), jnp.float32)
    for k in range(3):
        m = jnp.min(vals, axis=1, keepdims=True)
        ssum = ssum + jnp.sqrt(m)
        if k < 2:
            sel = jnp.where(vals == m, ci, big)
            cmin = jnp.min(sel, axis=1, keepdims=True)
            vals = jnp.where(ci == cmin, jnp.inf, vals)
    dens = ssum * (1.0 / 3.0)

    acc = cvec_ref[...] + cd * vdist_ref[...] + dens * vdens_ref[...]
    acc = acc + rel[:, 0:1] * mrel_ref[0:1, :]
    acc = acc + rel[:, 1:2] * mrel_ref[1:2, :]
    acc = acc + rel[:, 2:3] * mrel_ref[2:3, :]
    out_ref[0] = acc


def kernel(points, W_rel, b_rel, W_dist, b_dist, W_dens, b_dens, W_out, b_out):
    mrel = W_rel @ W_out[:SUB]
    vdist = W_dist @ W_out[SUB:2 * SUB]
    vdens = W_dens @ W_out[2 * SUB:]
    cvec = (b_rel @ W_out[:SUB] + b_dist @ W_out[SUB:2 * SUB]
            + b_dens @ W_out[2 * SUB:] + b_out)[None, :]

    pts_t = jnp.transpose(points, (0, 2, 1))
    pts_r = lax.reduce_precision(pts_t, exponent_bits=8, mantissa_bits=7)
    xr, yr, zr = pts_r[:, 0], pts_r[:, 1], pts_r[:, 2]
    x2 = jnp.sum(pts_t * pts_t, axis=1)

    cand_raw = _knn_sc(xr, yr, zr, x2)
    cand = cand_raw.reshape(B, N, CAND)

    return pl.pallas_call(
        _tc_body,
        grid=(B,),
        in_specs=[
            pl.BlockSpec((1, N, 3), lambda b: (b, 0, 0)),
            pl.BlockSpec((1, N, CAND), lambda b: (b, 0, 0)),
            pl.BlockSpec((3, EMBED_DIM), lambda b: (0, 0)),
            pl.BlockSpec((1, EMBED_DIM), lambda b: (0, 0)),
            pl.BlockSpec((1, EMBED_DIM), lambda b: (0, 0)),
            pl.BlockSpec((1, EMBED_DIM), lambda b: (0, 0)),
        ],
        out_specs=pl.BlockSpec((1, N, EMBED_DIM), lambda b: (b, 0, 0)),
        out_shape=jax.ShapeDtypeStruct((B, N, EMBED_DIM), jnp.float32),
    )(points, cand, mrel, vdist, vdens, cvec)


# speedup vs baseline: 1.5182x; 1.0001x over previous
import functools

import jax
import jax.numpy as jnp
from jax import lax
from jax.experimental import pallas as pl
from jax.experimental.pallas import tpu as pltpu
from jax.experimental.pallas import tpu_sc as plsc

EMBED_DIM = 384
SUB = EMBED_DIM // 3
B, N = 16, 2048
HALF = N // 2
ROWS = 256
T = N // ROWS
L = 16
CHUNKS = N // L
K3 = 3
CAND = K3 * L

_mesh = plsc.VectorSubcoreMesh(core_axis_name="c", subcore_axis_name="s")


@functools.partial(
    pl.kernel,
    out_type=jax.ShapeDtypeStruct((B, 2, HALF * CAND), jnp.float32),
    mesh=_mesh,
    scratch_types=[
        pltpu.VMEM((N,), jnp.float32),
        pltpu.VMEM((N,), jnp.float32),
        pltpu.VMEM((N,), jnp.float32),
        pltpu.VMEM((N,), jnp.float32),
        pltpu.VMEM((HALF * CAND,), jnp.float32),
    ],
)
def _knn_sc(xr_hbm, yr_hbm, zr_hbm, x2_hbm, out_hbm,
            xr_v, yr_v, zr_v, x2_v, om_v):
    half = lax.axis_index("c")
    batch = lax.axis_index("s")

    pltpu.sync_copy(xr_hbm.at[batch], xr_v)
    pltpu.sync_copy(yr_hbm.at[batch], yr_v)
    pltpu.sync_copy(zr_hbm.at[batch], zr_v)
    pltpu.sync_copy(x2_hbm.at[batch], x2_v)

    iota = lax.broadcasted_iota(jnp.int32, (L,), 0)
    inf_v = jnp.full((L,), jnp.inf, jnp.float32)
    rowbase = half * HALF

    R = 4

    def multi_proc(r_loc, cs, rr, qs):
        bs = [[jnp.full((L,), -2.0 * q, jnp.float32) for q in t] for t in qs]

        def insert(t, m1, m2, m3):
            lo = jnp.minimum(m1, t)
            hi = jnp.maximum(m1, t)
            lo2 = jnp.minimum(m2, hi)
            hi2 = jnp.maximum(m2, hi)
            return lo, lo2, jnp.minimum(m3, hi2)

        def chunk_all(ci, carry, masked):
            sl = pl.ds(ci * L, L)
            vx, vy, vz, v2 = xr_v[sl], yr_v[sl], zr_v[sl], x2_v[sl]
            out = []
            for i in range(R):
                b = bs[i]
                t = v2 + ((vx * b[0] + vy * b[1]) + vz * b[2])
                if masked:
                    t = jnp.where(iota == rr + i, inf_v, t)
                out.extend(insert(t, *carry[3 * i:3 * i + 3]))
            return tuple(out)

        def cbody(ci, carry):
            return chunk_all(ci, carry, False)

        m = plsc.parallel_loop(0, cs, 1, unroll=2,
                               carry=(inf_v,) * (3 * R))(cbody)
        m = chunk_all(cs, m, True)
        m = plsc.parallel_loop(cs + 1, CHUNKS, 1, unroll=2, carry=m)(cbody)
        for i in range(R):
            off = (r_loc + i) * CAND
            for k in range(K3):
                om_v[pl.ds(off + k * L, L)] = m[3 * i + k]

    def gbody(g, _):
        lb = g * L
        sl = pl.ds(rowbase + lb, L)
        vxr, vyr, vzr = xr_v[sl], yr_v[sl], zr_v[sl]
        cs = half * (HALF // L) + g
        for rr in range(0, L, R):
            multi_proc(lb + rr, cs, rr,
                       [(vxr[rr + i], vyr[rr + i], vzr[rr + i])
                        for i in range(R)])
        return 0

    lax.fori_loop(0, HALF // L, gbody, 0)
    pltpu.sync_copy(om_v, out_hbm.at[batch, half])


def _tc_body(pts_ref, cand_ref, mrel_ref, vdist_ref, vdens_ref, cvec_ref,
             out_ref):
    rows = pts_ref[0]

    cen = jnp.mean(rows, axis=0, keepdims=True)
    rel = rows - cen
    cd = jnp.sqrt(jnp.sum(rel * rel, axis=1, keepdims=True))

    x2r = jnp.sum(rows * rows, axis=1, keepdims=True)
    vals = jnp.maximum(cand_ref[0] + x2r, 0.0)
    ci = lax.broadcasted_iota(jnp.int32, (N, CAND), 1).astype(jnp.float32)
    big = jnp.float32(1e9)
    ssum = jnp.zeros((N, 1), jnp.float32)
    for k in range(3):
        m = jnp.min(vals, axis=1, keepdims=True)
        ssum = ssum + jnp.sqrt(m)
        if k < 2:
            sel = jnp.where(vals == m, ci, big)
            cmin = jnp.min(sel, axis=1, keepdims=True)
            vals = jnp.where(ci == cmin, jnp.inf, vals)
    dens = ssum * (1.0 / 3.0)

    acc = cvec_ref[...] + cd * vdist_ref[...] + dens * vdens_ref[...]
    acc = acc + rel[:, 0:1] * mrel_ref[0:1, :]
    acc = acc + rel[:, 1:2] * mrel_ref[1:2, :]
    acc = acc + rel[:, 2:3] * mrel_ref[2:3, :]
    out_ref[0] = acc


def kernel(points, W_rel, b_rel, W_dist, b_dist, W_dens, b_dens, W_out, b_out):
    mrel = W_rel @ W_out[:SUB]
    vdist = W_dist @ W_out[SUB:2 * SUB]
    vdens = W_dens @ W_out[2 * SUB:]
    cvec = (b_rel @ W_out[:SUB] + b_dist @ W_out[SUB:2 * SUB]
            + b_dens @ W_out[2 * SUB:] + b_out)[None, :]

    pts_t = jnp.transpose(points, (0, 2, 1))
    pts_r = lax.reduce_precision(pts_t, exponent_bits=8, mantissa_bits=7)
    xr, yr, zr = pts_r[:, 0], pts_r[:, 1], pts_r[:, 2]
    x2 = jnp.sum(pts_t * pts_t, axis=1)

    cand_raw = _knn_sc(xr, yr, zr, x2)
    cand = cand_raw.reshape(B, N, CAND)

    return pl.pallas_call(
        _tc_body,
        grid=(B,),
        in_specs=[
            pl.BlockSpec((1, N, 3), lambda b: (b, 0, 0)),
            pl.BlockSpec((1, N, CAND), lambda b: (b, 0, 0)),
            pl.BlockSpec((3, EMBED_DIM), lambda b: (0, 0)),
            pl.BlockSpec((1, EMBED_DIM), lambda b: (0, 0)),
            pl.BlockSpec((1, EMBED_DIM), lambda b: (0, 0)),
            pl.BlockSpec((1, EMBED_DIM), lambda b: (0, 0)),
        ],
        out_specs=pl.BlockSpec((1, N, EMBED_DIM), lambda b: (b, 0, 0)),
        out_shape=jax.ShapeDtypeStruct((B, N, EMBED_DIM), jnp.float32),
    )(points, cand, mrel, vdist, vdens, cvec)
